# baseline (speedup 1.0000x reference)
import functools

import jax
import jax.numpy as jnp
from jax import lax
from jax.experimental import pallas as pl

def _fused_body(nkb, kb_size, flat_ref, protos_ref, closest_ref, counts_ref,
                sums_ref):
    i = pl.program_id(0)

    @pl.when(i == 0)
    def _init():
        counts_ref[...] = jnp.zeros_like(counts_ref)
        sums_ref[...] = jnp.zeros_like(sums_ref)

    z = flat_ref[...]
    m = z.shape[0]
    z2 = jnp.sum(z * z, axis=1, keepdims=True)

    best_val = jnp.full((m,), jnp.inf, jnp.float32)
    best_idx = jnp.zeros((m,), jnp.int32)
    for kb in range(nkb):
        p = protos_ref[pl.ds(kb * kb_size, kb_size), :]
        c2 = jnp.sum(p * p, axis=1)
        cross = lax.dot_general(z, p, (((1,), (1,)), ((), ())),
                                preferred_element_type=jnp.float32,
                                precision=lax.Precision.DEFAULT)
        dist = jnp.maximum(z2 + c2[None, :] - 2.0 * cross, 0.0)
        bmin = jnp.min(dist, axis=1, keepdims=True)
        iota = lax.broadcasted_iota(jnp.int32, dist.shape, 1)
        barg = jnp.min(jnp.where(dist <= bmin, iota, 2**30), axis=1)
        bminv = bmin[:, 0]
        upd = bminv < best_val
        best_val = jnp.where(upd, bminv, best_val)
        best_idx = jnp.where(upd, barg + kb * kb_size, best_idx)

    closest_ref[0, 0, :] = best_idx

    for kb in range(nkb):
        iota = lax.broadcasted_iota(jnp.int32, (m, kb_size), 1) + kb * kb_size
        maskf = (best_idx[:, None] == iota).astype(jnp.float32)
        counts_ref[kb, :] += jnp.sum(maskf, axis=0)
        sums_ref[pl.ds(kb * kb_size, kb_size), :] += lax.dot_general(
            maskf, z, (((0,), (0,)), ((), ())),
            preferred_element_type=jnp.float32,
            precision=lax.Precision.HIGHEST)


def kernel(batch, protos):
    b, n, d = batch.shape
    k = protos.shape[0]
    npts = b * n
    flat = batch.reshape(npts, d)

    m = min(512, npts)
    kb_size = min(512, k)
    nblk = npts // m
    nkb = k // kb_size

    closest3, counts2, sums = pl.pallas_call(
        functools.partial(_fused_body, nkb, kb_size),
        grid=(nblk,),
        in_specs=[
            pl.BlockSpec((m, d), lambda i: (i, 0)),
            pl.BlockSpec((k, d), lambda i: (0, 0)),
        ],
        out_specs=[
            pl.BlockSpec((1, 1, m), lambda i: (i, 0, 0)),
            pl.BlockSpec((nkb, kb_size), lambda i: (0, 0)),
            pl.BlockSpec((k, d), lambda i: (0, 0)),
        ],
        out_shape=[
            jax.ShapeDtypeStruct((nblk, 1, m), jnp.int32),
            jax.ShapeDtypeStruct((nkb, kb_size), jnp.float32),
            jax.ShapeDtypeStruct((k, d), jnp.float32),
        ],
    )(flat, protos)

    closest = closest3.reshape(b, n)
    counts = counts2.reshape(k)
    return closest, sums, counts


# speedup vs baseline: 1.7239x; 1.7239x over previous
import functools

import jax
import jax.numpy as jnp
from jax import lax
from jax.experimental import pallas as pl
from jax.experimental.pallas import tpu as pltpu
from jax.experimental.pallas import tpu_sc as plsc

NC, NS = 2, 16


def _argmin_body(nkb, kb_size, flat_ref, protos_ref, closest_ref, counts_ref):
    i = pl.program_id(0)

    @pl.when(i == 0)
    def _init():
        counts_ref[...] = jnp.zeros_like(counts_ref)

    z = flat_ref[...]
    m = z.shape[0]
    z2 = jnp.sum(z * z, axis=1, keepdims=True)

    best_val = jnp.full((m,), jnp.inf, jnp.float32)
    best_idx = jnp.zeros((m,), jnp.int32)
    for kb in range(nkb):
        p = protos_ref[pl.ds(kb * kb_size, kb_size), :]
        c2 = jnp.sum(p * p, axis=1)
        cross = lax.dot_general(z, p, (((1,), (1,)), ((), ())),
                                preferred_element_type=jnp.float32,
                                precision=lax.Precision.DEFAULT)
        dist = jnp.maximum(z2 + c2[None, :] - 2.0 * cross, 0.0)
        bmin = jnp.min(dist, axis=1, keepdims=True)
        iota = lax.broadcasted_iota(jnp.int32, dist.shape, 1)
        barg = jnp.min(jnp.where(dist <= bmin, iota, 2**30), axis=1)
        bminv = bmin[:, 0]
        upd = bminv < best_val
        best_val = jnp.where(upd, bminv, best_val)
        best_idx = jnp.where(upd, barg + kb * kb_size, best_idx)

    closest_ref[0, 0, :] = best_idx

    for kb in range(nkb):
        iota = lax.broadcasted_iota(jnp.int32, (m, kb_size), 1) + kb * kb_size
        maskf = (best_idx[:, None] == iota).astype(jnp.float32)
        counts_ref[kb, :] += jnp.sum(maskf, axis=0)


def _sc_sums_body(npts, k, dc, chunk, flat_a, flat_b, idx_hbm,
                  out_a, out_b, idx_v, data_v, zero_v, table):
    c = lax.axis_index("c")
    s = lax.axis_index("s")
    pts_per_tile = npts // NS
    nchunks = pts_per_tile // chunk
    zrows = zero_v.shape[0]
    rows_per_tile = k // NS

    def _zrow(r, carry):
        for j in range(dc // 16):
            zero_v[r, pl.ds(j * 16, 16)] = jnp.zeros((16,), jnp.float32)
        return carry
    lax.fori_loop(0, zrows, _zrow, 0)
    for q in range(rows_per_tile // zrows):
        pltpu.sync_copy(zero_v, table.at[pl.ds(s * rows_per_tile + q * zrows,
                                               zrows)])
    pltpu.sync_copy(idx_hbm.at[s], idx_v)
    plsc.subcore_barrier()

    base = s * pts_per_tile
    for j in range(nchunks):
        rows = pl.ds(base + j * chunk, chunk)

        @pl.when(c == 0)
        def _lda():
            pltpu.sync_copy(flat_a.at[rows], data_v)

        @pl.when(c == 1)
        def _ldb():
            pltpu.sync_copy(flat_b.at[rows], data_v)

        pltpu.sync_copy(data_v, table.at[idx_v.at[j]], add=True)
    plsc.subcore_barrier()

    stripe = pl.ds(s * rows_per_tile, rows_per_tile)

    @pl.when(c == 0)
    def _sta():
        pltpu.sync_copy(table.at[stripe], out_a.at[stripe])

    @pl.when(c == 1)
    def _stb():
        pltpu.sync_copy(table.at[stripe], out_b.at[stripe])


def kernel(batch, protos):
    b, n, d = batch.shape
    k = protos.shape[0]
    npts = b * n
    flat = batch.reshape(npts, d)

    m = min(512, npts)
    kb_size = min(512, k)
    nblk = npts // m
    nkb = k // kb_size

    closest3, counts2 = pl.pallas_call(
        functools.partial(_argmin_body, nkb, kb_size),
        grid=(nblk,),
        in_specs=[
            pl.BlockSpec((m, d), lambda i: (i, 0)),
            pl.BlockSpec((k, d), lambda i: (0, 0)),
        ],
        out_specs=[
            pl.BlockSpec((1, 1, m), lambda i: (i, 0, 0)),
            pl.BlockSpec((nkb, kb_size), lambda i: (0, 0)),
        ],
        out_shape=[
            jax.ShapeDtypeStruct((nblk, 1, m), jnp.int32),
            jax.ShapeDtypeStruct((nkb, kb_size), jnp.float32),
        ],
    )(flat, protos)

    closest = closest3.reshape(b, n)
    counts = counts2.reshape(k)

    dc = d // NC
    chunk = 128
    idx3 = closest3.reshape(NS, npts // NS // chunk, chunk)
    flat_a = flat[:, :dc]
    flat_b = flat[:, dc:]

    mesh = plsc.VectorSubcoreMesh(core_axis_name="c", subcore_axis_name="s")
    out_a, out_b = pl.kernel(
        functools.partial(_sc_sums_body, npts, k, dc, chunk),
        out_type=[jax.ShapeDtypeStruct((k, dc), jnp.float32),
                  jax.ShapeDtypeStruct((k, dc), jnp.float32)],
        mesh=mesh,
        scratch_types=[
            pltpu.VMEM((npts // NS // chunk, chunk), jnp.int32),
            pltpu.VMEM((chunk, dc), jnp.float32),
            pltpu.VMEM((128, dc), jnp.float32),
            pltpu.VMEM_SHARED((k, dc), jnp.float32),
        ],
    )(flat_a, flat_b, idx3)
    sums = jnp.concatenate([out_a, out_b], axis=1)

    return closest, sums, counts


# speedup vs baseline: 2.0811x; 1.2072x over previous
import functools

import jax
import jax.numpy as jnp
from jax import lax
from jax.experimental import pallas as pl
from jax.experimental.pallas import tpu as pltpu
from jax.experimental.pallas import tpu_sc as plsc

NC, NS = 2, 16


def _argmin_body(nkb, kb_size, flat_ref, protos_ref, closest_ref):
    z = flat_ref[...]
    m = z.shape[0]
    z2 = jnp.sum(z * z, axis=1, keepdims=True)

    best_val = jnp.full((m,), jnp.inf, jnp.float32)
    best_idx = jnp.zeros((m,), jnp.float32)
    for kb in range(nkb):
        p = protos_ref[pl.ds(kb * kb_size, kb_size), :]
        c2 = jnp.sum(p * p, axis=1)
        cross = lax.dot_general(z, p, (((1,), (1,)), ((), ())),
                                preferred_element_type=jnp.float32,
                                precision=lax.Precision.DEFAULT)
        x = z2 + c2[None, :] - 2.0 * cross
        bmin = jnp.maximum(jnp.min(x, axis=1, keepdims=True), 0.0)
        iota = lax.broadcasted_iota(jnp.int32, x.shape, 1).astype(jnp.float32)
        barg = jnp.min(jnp.where(x <= bmin, iota, 3e9), axis=1)
        bminv = bmin[:, 0]
        upd = bminv < best_val
        best_val = jnp.where(upd, bminv, best_val)
        best_idx = jnp.where(upd, barg + float(kb * kb_size), best_idx)

    closest_ref[0, 0, :] = best_idx.astype(jnp.int32)


def _counts_body(nkb, kb_size, closest_ref, counts_ref):
    i = pl.program_id(0)

    @pl.when(i == 0)
    def _init():
        counts_ref[...] = jnp.zeros_like(counts_ref)

    best_idx = closest_ref[0, 0, :]
    m = best_idx.shape[0]
    for kb in range(nkb):
        iota = lax.broadcasted_iota(jnp.int32, (m, kb_size), 1) + kb * kb_size
        maskf = (best_idx[:, None] == iota).astype(jnp.float32)
        counts_ref[kb, :] += jnp.sum(maskf, axis=0)


def _sc_sums_body(npts, k, dc, chunk, flat_a, flat_b, idx_hbm,
                  out_a, out_b, idx_v, data_v, zero_v, table):
    c = lax.axis_index("c")
    s = lax.axis_index("s")
    pts_per_tile = npts // NS
    nchunks = pts_per_tile // chunk
    zrows = zero_v.shape[0]
    rows_per_tile = k // NS

    def _zrow(r, carry):
        for j in range(dc // 16):
            zero_v[r, pl.ds(j * 16, 16)] = jnp.zeros((16,), jnp.float32)
        return carry
    lax.fori_loop(0, zrows, _zrow, 0)
    for q in range(rows_per_tile // zrows):
        pltpu.sync_copy(zero_v, table.at[pl.ds(s * rows_per_tile + q * zrows,
                                               zrows)])
    pltpu.sync_copy(idx_hbm.at[s], idx_v)
    plsc.subcore_barrier()

    base = s * pts_per_tile
    for j in range(nchunks):
        rows = pl.ds(base + j * chunk, chunk)

        @pl.when(c == 0)
        def _lda():
            pltpu.sync_copy(flat_a.at[rows], data_v)

        @pl.when(c == 1)
        def _ldb():
            pltpu.sync_copy(flat_b.at[rows], data_v)

        pltpu.sync_copy(data_v, table.at[idx_v.at[j]], add=True)

    plsc.subcore_barrier()

    stripe = pl.ds(s * rows_per_tile, rows_per_tile)

    @pl.when(c == 0)
    def _sta():
        pltpu.sync_copy(table.at[stripe], out_a.at[stripe])

    @pl.when(c == 1)
    def _stb():
        pltpu.sync_copy(table.at[stripe], out_b.at[stripe])


def kernel(batch, protos):
    b, n, d = batch.shape
    k = protos.shape[0]
    npts = b * n
    flat = batch.reshape(npts, d)

    m = min(512, npts)
    kb_size = min(512, k)
    nblk = npts // m
    nkb = k // kb_size

    closest3 = pl.pallas_call(
        functools.partial(_argmin_body, nkb, kb_size),
        grid=(nblk,),
        in_specs=[
            pl.BlockSpec((m, d), lambda i: (i, 0)),
            pl.BlockSpec((k, d), lambda i: (0, 0)),
        ],
        out_specs=pl.BlockSpec((1, 1, m), lambda i: (i, 0, 0)),
        out_shape=jax.ShapeDtypeStruct((nblk, 1, m), jnp.int32),
    )(flat, protos)

    closest = closest3.reshape(b, n)

    dc = d // NC
    chunk = 128
    idx3 = closest3.reshape(NS, npts // NS // chunk, chunk)
    flat_a = flat[:, :dc]
    flat_b = flat[:, dc:]

    mesh = plsc.VectorSubcoreMesh(core_axis_name="c", subcore_axis_name="s")
    out_a, out_b = pl.kernel(
        functools.partial(_sc_sums_body, npts, k, dc, chunk),
        out_type=[jax.ShapeDtypeStruct((k, dc), jnp.float32),
                  jax.ShapeDtypeStruct((k, dc), jnp.float32)],
        mesh=mesh,
        scratch_types=[
            pltpu.VMEM((npts // NS // chunk, chunk), jnp.int32),
            pltpu.VMEM((chunk, dc), jnp.float32),
            pltpu.VMEM((128, dc), jnp.float32),
            pltpu.VMEM_SHARED((k, dc), jnp.float32),
        ],
    )(flat_a, flat_b, idx3)
    sums = jnp.concatenate([out_a, out_b], axis=1)

    counts2 = pl.pallas_call(
        functools.partial(_counts_body, nkb, kb_size),
        grid=(nblk,),
        in_specs=[pl.BlockSpec((1, 1, m), lambda i: (i, 0, 0))],
        out_specs=pl.BlockSpec((nkb, kb_size), lambda i: (0, 0)),
        out_shape=jax.ShapeDtypeStruct((nkb, kb_size), jnp.float32),
    )(closest3)
    counts = counts2.reshape(k)

    return closest, sums, counts


# speedup vs baseline: 2.8291x; 1.3594x over previous
import functools

import jax
import jax.numpy as jnp
from jax import lax
from jax.experimental import pallas as pl
from jax.experimental.pallas import tpu as pltpu
from jax.experimental.pallas import tpu_sc as plsc

NC, NS = 2, 16


def _prep_body(nkb, kb_size, protos_ref, q_ref, c2_ref):
    for kb in range(nkb):
        sl = pl.ds(kb * kb_size, kb_size)
        p = protos_ref[sl, :]
        q_ref[sl, :] = -2.0 * p
        c2_ref[0, sl] = jnp.sum(p * p, axis=1)


def _argmin_body(nkb, kb_size, flat_ref, q_ref, c2_ref, closest_ref):
    z = flat_ref[...]
    m = z.shape[0]
    z2 = jnp.sum(z * z, axis=1, keepdims=True)

    best_val = jnp.full((m,), jnp.inf, jnp.float32)
    best_idx = jnp.zeros((m,), jnp.float32)
    for kb in range(nkb):
        q = q_ref[pl.ds(kb * kb_size, kb_size), :]
        c2 = c2_ref[0, pl.ds(kb * kb_size, kb_size)]
        crossn2 = lax.dot_general(z, q, (((1,), (1,)), ((), ())),
                                  preferred_element_type=jnp.float32,
                                  precision=lax.Precision.DEFAULT)
        x = (z2 + c2[None, :]) + crossn2
        bmin = jnp.maximum(jnp.min(x, axis=1, keepdims=True), 0.0)
        iota = lax.broadcasted_iota(jnp.int32, x.shape, 1).astype(jnp.float32)
        barg = jnp.min(jnp.where(x <= bmin, iota, 3e9), axis=1)
        bminv = bmin[:, 0]
        upd = bminv < best_val
        best_val = jnp.where(upd, bminv, best_val)
        best_idx = jnp.where(upd, barg + float(kb * kb_size), best_idx)

    closest_ref[0, 0, :] = best_idx.astype(jnp.int32)


def _counts_body(nkb, kb_size, closest_ref, counts_ref):
    i = pl.program_id(0)

    @pl.when(i == 0)
    def _init():
        counts_ref[...] = jnp.zeros_like(counts_ref)

    best_idx = closest_ref[0, 0, :]
    m = best_idx.shape[0]
    for kb in range(nkb):
        iota = lax.broadcasted_iota(jnp.int32, (m, kb_size), 1) + kb * kb_size
        maskf = (best_idx[:, None] == iota).astype(jnp.float32)
        counts_ref[kb, :] += jnp.sum(maskf, axis=0)


def _sc_sums_body(npts, k, dc, chunk, flat_a, flat_b, idx_hbm,
                  out_a, out_b, idx_v, data_v, zero_v, table):
    c = lax.axis_index("c")
    s = lax.axis_index("s")
    pts_per_tile = npts // NS
    nchunks = pts_per_tile // chunk
    zrows = zero_v.shape[0]
    rows_per_tile = k // NS

    def _zrow(r, carry):
        for j in range(dc // 16):
            zero_v[r, pl.ds(j * 16, 16)] = jnp.zeros((16,), jnp.float32)
        return carry
    lax.fori_loop(0, zrows, _zrow, 0)
    for q in range(rows_per_tile // zrows):
        pltpu.sync_copy(zero_v, table.at[pl.ds(s * rows_per_tile + q * zrows,
                                               zrows)])
    pltpu.sync_copy(idx_hbm.at[s], idx_v)
    plsc.subcore_barrier()

    base = s * pts_per_tile
    for j in range(nchunks):
        rows = pl.ds(base + j * chunk, chunk)

        @pl.when(c == 0)
        def _lda():
            pltpu.sync_copy(flat_a.at[rows], data_v)

        @pl.when(c == 1)
        def _ldb():
            pltpu.sync_copy(flat_b.at[rows], data_v)

        pltpu.sync_copy(data_v, table.at[idx_v.at[j]], add=True)

    plsc.subcore_barrier()

    stripe = pl.ds(s * rows_per_tile, rows_per_tile)

    @pl.when(c == 0)
    def _sta():
        pltpu.sync_copy(table.at[stripe], out_a.at[stripe])

    @pl.when(c == 1)
    def _stb():
        pltpu.sync_copy(table.at[stripe], out_b.at[stripe])


def kernel(batch, protos):
    b, n, d = batch.shape
    k = protos.shape[0]
    npts = b * n
    flat = batch.reshape(npts, d)

    m = min(512, npts)
    kb_size = min(512, k)
    nblk = npts // m
    nkb = k // kb_size

    q, c2 = pl.pallas_call(
        functools.partial(_prep_body, nkb, kb_size),
        in_specs=[pl.BlockSpec((k, d), lambda: (0, 0))],
        out_specs=[pl.BlockSpec((k, d), lambda: (0, 0)),
                   pl.BlockSpec((1, k), lambda: (0, 0))],
        out_shape=[jax.ShapeDtypeStruct((k, d), jnp.float32),
                   jax.ShapeDtypeStruct((1, k), jnp.float32)],
    )(protos)

    closest3 = pl.pallas_call(
        functools.partial(_argmin_body, nkb, kb_size),
        grid=(nblk,),
        in_specs=[
            pl.BlockSpec((m, d), lambda i: (i, 0)),
            pl.BlockSpec((k, d), lambda i: (0, 0)),
            pl.BlockSpec((1, k), lambda i: (0, 0)),
        ],
        out_specs=pl.BlockSpec((1, 1, m), lambda i: (i, 0, 0)),
        out_shape=jax.ShapeDtypeStruct((nblk, 1, m), jnp.int32),
    )(flat, q, c2)

    closest = closest3.reshape(b, n)

    dc = d // NC
    chunk = 128
    idx3 = closest3.reshape(NS, npts // NS // chunk, chunk)
    flat_a = flat[:, :dc]
    flat_b = flat[:, dc:]

    mesh = plsc.VectorSubcoreMesh(core_axis_name="c", subcore_axis_name="s")
    out_a, out_b = pl.kernel(
        functools.partial(_sc_sums_body, npts, k, dc, chunk),
        out_type=[jax.ShapeDtypeStruct((k, dc), jnp.float32),
                  jax.ShapeDtypeStruct((k, dc), jnp.float32)],
        mesh=mesh,
        scratch_types=[
            pltpu.VMEM((npts // NS // chunk, chunk), jnp.int32),
            pltpu.VMEM((chunk, dc), jnp.float32),
            pltpu.VMEM((128, dc), jnp.float32),
            pltpu.VMEM_SHARED((k, dc), jnp.float32),
        ],
    )(flat_a, flat_b, idx3)
    sums = jnp.concatenate([out_a, out_b], axis=1)

    counts2 = pl.pallas_call(
        functools.partial(_counts_body, nkb, kb_size),
        grid=(nblk,),
        in_specs=[pl.BlockSpec((1, 1, m), lambda i: (i, 0, 0))],
        out_specs=pl.BlockSpec((nkb, kb_size), lambda i: (0, 0)),
        out_shape=jax.ShapeDtypeStruct((nkb, kb_size), jnp.float32),
    )(closest3)
    counts = counts2.reshape(k)

    return closest, sums, counts


# speedup vs baseline: 2.9532x; 1.0439x over previous
import functools

import jax
import jax.numpy as jnp
from jax import lax
from jax.experimental import pallas as pl
from jax.experimental.pallas import tpu as pltpu
from jax.experimental.pallas import tpu_sc as plsc

NC, NS = 2, 16


def _prep_body(nkb, kb_size, protos_ref, q_ref, c2_ref):
    for kb in range(nkb):
        sl = pl.ds(kb * kb_size, kb_size)
        p = protos_ref[sl, :]
        q_ref[sl, :] = -2.0 * p
        c2_ref[0, sl] = jnp.sum(p * p, axis=1)


def _argmin_body(nkb, kb_size, flat_ref, q_ref, c2_ref, closest_ref):
    z = flat_ref[...]
    m = z.shape[0]
    z2 = jnp.sum(z * z, axis=1, keepdims=True)

    best_val = jnp.full((m,), jnp.inf, jnp.float32)
    best_idx = jnp.zeros((m,), jnp.float32)
    for kb in range(nkb):
        q = q_ref[pl.ds(kb * kb_size, kb_size), :]
        c2 = c2_ref[0, pl.ds(kb * kb_size, kb_size)]
        crossn2 = lax.dot_general(z, q, (((1,), (1,)), ((), ())),
                                  preferred_element_type=jnp.float32,
                                  precision=lax.Precision.DEFAULT)
        x = (z2 + c2[None, :]) + crossn2
        bmin = jnp.maximum(jnp.min(x, axis=1, keepdims=True), 0.0)
        iota = lax.broadcasted_iota(jnp.int32, x.shape, 1).astype(jnp.float32)
        barg = jnp.min(jnp.where(x <= bmin, iota, 3e9), axis=1)
        bminv = bmin[:, 0]
        upd = bminv < best_val
        best_val = jnp.where(upd, bminv, best_val)
        best_idx = jnp.where(upd, barg + float(kb * kb_size), best_idx)

    closest_ref[0, 0, :] = best_idx.astype(jnp.int32)


def _counts_body(nkb, kb_size, closest_ref, counts_ref):
    i = pl.program_id(0)

    @pl.when(i == 0)
    def _init():
        counts_ref[...] = jnp.zeros_like(counts_ref)

    best_idx = closest_ref[0, 0, :]
    m = best_idx.shape[0]
    for kb in range(nkb):
        iota = lax.broadcasted_iota(jnp.int32, (m, kb_size), 1) + kb * kb_size
        maskf = (best_idx[:, None] == iota).astype(jnp.float32)
        counts_ref[kb, :] += jnp.sum(maskf, axis=0)


def _sc_sums_body(npts, k, dc, chunk, flat_a, flat_b, idx_hbm,
                  out_a, out_b, idx_v, data_v, zero_v, table):
    c = lax.axis_index("c")
    s = lax.axis_index("s")
    pts_per_tile = npts // NS
    nchunks = pts_per_tile // chunk
    zrows = zero_v.shape[0]
    rows_per_tile = k // NS

    def _zrow(r, carry):
        for j in range(dc // 16):
            zero_v[r, pl.ds(j * 16, 16)] = jnp.zeros((16,), jnp.float32)
        return carry
    lax.fori_loop(0, zrows, _zrow, 0)
    for q in range(rows_per_tile // zrows):
        pltpu.sync_copy(zero_v, table.at[pl.ds(s * rows_per_tile + q * zrows,
                                               zrows)])
    pltpu.sync_copy(idx_hbm.at[s], idx_v)
    plsc.subcore_barrier()

    base = s * pts_per_tile
    for j in range(nchunks):
        rows = pl.ds(base + j * chunk, chunk)

        @pl.when(c == 0)
        def _lda():
            pltpu.sync_copy(flat_a.at[rows], data_v)

        @pl.when(c == 1)
        def _ldb():
            pltpu.sync_copy(flat_b.at[rows], data_v)

        pltpu.sync_copy(data_v, table.at[idx_v.at[j]], add=True)

    plsc.subcore_barrier()

    stripe = pl.ds(s * rows_per_tile, rows_per_tile)

    @pl.when(c == 0)
    def _sta():
        pltpu.sync_copy(table.at[stripe], out_a.at[stripe])

    @pl.when(c == 1)
    def _stb():
        pltpu.sync_copy(table.at[stripe], out_b.at[stripe])


def kernel(batch, protos):
    b, n, d = batch.shape
    k = protos.shape[0]
    npts = b * n
    flat = batch.reshape(npts, d)

    m = min(1024, npts)
    kb_size = min(512, k)
    nblk = npts // m
    nkb = k // kb_size

    q, c2 = pl.pallas_call(
        functools.partial(_prep_body, nkb, kb_size),
        in_specs=[pl.BlockSpec((k, d), lambda: (0, 0))],
        out_specs=[pl.BlockSpec((k, d), lambda: (0, 0)),
                   pl.BlockSpec((1, k), lambda: (0, 0))],
        out_shape=[jax.ShapeDtypeStruct((k, d), jnp.float32),
                   jax.ShapeDtypeStruct((1, k), jnp.float32)],
    )(protos)

    closest3 = pl.pallas_call(
        functools.partial(_argmin_body, nkb, kb_size),
        grid=(nblk,),
        in_specs=[
            pl.BlockSpec((m, d), lambda i: (i, 0)),
            pl.BlockSpec((k, d), lambda i: (0, 0)),
            pl.BlockSpec((1, k), lambda i: (0, 0)),
        ],
        out_specs=pl.BlockSpec((1, 1, m), lambda i: (i, 0, 0)),
        out_shape=jax.ShapeDtypeStruct((nblk, 1, m), jnp.int32),
    )(flat, q, c2)

    closest = closest3.reshape(b, n)

    dc = d // NC
    chunk = 128
    idx3 = closest3.reshape(NS, npts // NS // chunk, chunk)
    flat_a = flat[:, :dc]
    flat_b = flat[:, dc:]

    mesh = plsc.VectorSubcoreMesh(core_axis_name="c", subcore_axis_name="s")
    out_a, out_b = pl.kernel(
        functools.partial(_sc_sums_body, npts, k, dc, chunk),
        out_type=[jax.ShapeDtypeStruct((k, dc), jnp.float32),
                  jax.ShapeDtypeStruct((k, dc), jnp.float32)],
        mesh=mesh,
        scratch_types=[
            pltpu.VMEM((npts // NS // chunk, chunk), jnp.int32),
            pltpu.VMEM((chunk, dc), jnp.float32),
            pltpu.VMEM((128, dc), jnp.float32),
            pltpu.VMEM_SHARED((k, dc), jnp.float32),
        ],
    )(flat_a, flat_b, idx3)
    sums = jnp.concatenate([out_a, out_b], axis=1)

    counts2 = pl.pallas_call(
        functools.partial(_counts_body, nkb, kb_size),
        grid=(nblk,),
        in_specs=[pl.BlockSpec((1, 1, m), lambda i: (i, 0, 0))],
        out_specs=pl.BlockSpec((nkb, kb_size), lambda i: (0, 0)),
        out_shape=jax.ShapeDtypeStruct((nkb, kb_size), jnp.float32),
    )(closest3)
    counts = counts2.reshape(k)

    return closest, sums, counts
